# Initial kernel scaffold; baseline (speedup 1.0000x reference)
import jax
import jax.numpy as jnp
from jax.experimental import pallas as pl


def kernel(x_z, W1, b1, W2, b2, W3, b3, temperature):
    raise NotImplementedError("write your pallas kernel here")


# speedup vs baseline: 1.5687x; 1.5687x over previous
import jax
import jax.numpy as jnp
from jax.experimental import pallas as pl

T = 2048
E = 8
D = 1024
H = 512

BT = 256


def _gate_kernel(x_ref, w1_ref, b1_ref, w2_ref, b2_ref, w3_ref, scal_ref,
                 soft_ref, hard_ref):
    h = jnp.dot(x_ref[...], w1_ref[...], preferred_element_type=jnp.float32)
    h = jnp.maximum(h + b1_ref[...], 0.0)
    h = jnp.dot(h, w2_ref[...], preferred_element_type=jnp.float32)
    h = jnp.maximum(h + b2_ref[...], 0.0)
    logit = jnp.dot(h, w3_ref[...], preferred_element_type=jnp.float32)
    b3 = scal_ref[0, 0]
    inv_t = scal_ref[0, 1]
    y = (logit + b3) * inv_t
    y3 = y.reshape(BT, E, 1)
    m = jnp.max(y3, axis=1, keepdims=True)
    e = jnp.exp(y3 - m)
    s = jnp.sum(e, axis=1, keepdims=True)
    soft = e / s
    soft_ref[...] = soft
    ms = jnp.max(soft, axis=1, keepdims=True)
    ii = jax.lax.broadcasted_iota(jnp.int32, (BT, E, 1), 1)
    win = jnp.min(jnp.where(soft == ms, ii, E), axis=1, keepdims=True)
    hard_ref[...] = jnp.where(ii == win, 1.0, 0.0).astype(jnp.float32)


def kernel(x_z, W1, b1, W2, b2, W3, b3, temperature):
    x2d = x_z.reshape(T * E, D)
    b1r = b1.reshape(1, H)
    b2r = b2.reshape(1, H)
    scal = jnp.stack([b3[0], 1.0 / temperature]).reshape(1, 2).astype(jnp.float32)

    R = BT * E
    soft, hard = pl.pallas_call(
        _gate_kernel,
        grid=(T // BT,),
        in_specs=[
            pl.BlockSpec((R, D), lambda i: (i, 0)),
            pl.BlockSpec((D, H), lambda i: (0, 0)),
            pl.BlockSpec((1, H), lambda i: (0, 0)),
            pl.BlockSpec((H, H), lambda i: (0, 0)),
            pl.BlockSpec((1, H), lambda i: (0, 0)),
            pl.BlockSpec((H, 1), lambda i: (0, 0)),
            pl.BlockSpec((1, 2), lambda i: (0, 0)),
        ],
        out_specs=[
            pl.BlockSpec((BT, E, 1), lambda i: (i, 0, 0)),
            pl.BlockSpec((BT, E, 1), lambda i: (i, 0, 0)),
        ],
        out_shape=[
            jax.ShapeDtypeStruct((T, E, 1), jnp.float32),
            jax.ShapeDtypeStruct((T, E, 1), jnp.float32),
        ],
    )(x2d, W1, b1r, W2, b2r, W3, scal)
    return soft, hard


# speedup vs baseline: 2.1206x; 1.3518x over previous
import jax
import jax.numpy as jnp
from jax.experimental import pallas as pl

T = 2048
E = 8
D = 1024
H = 512

BT = 256


def _gate_kernel(x_ref, w1_ref, b1_ref, w2_ref, b2_ref, w3_ref, scal_ref,
                 soft_ref, hard_ref):
    h = jnp.dot(x_ref[...], w1_ref[...], preferred_element_type=jnp.float32)
    h = jnp.maximum(h + b1_ref[...], 0.0)
    h = jnp.dot(h, w2_ref[...], preferred_element_type=jnp.float32)
    h = jnp.maximum(h + b2_ref[...], 0.0)
    logit = jnp.dot(h, w3_ref[...], preferred_element_type=jnp.float32)
    b3 = scal_ref[0, 0]
    inv_t = scal_ref[0, 1]
    yt = logit.reshape(BT, E).T
    y = (yt + b3) * inv_t
    m = jnp.max(y, axis=0, keepdims=True)
    e = jnp.exp(y - m)
    s = jnp.sum(e, axis=0, keepdims=True)
    soft = e / s
    soft_ref[...] = soft
    ms = jnp.max(soft, axis=0, keepdims=True)
    ii = jax.lax.broadcasted_iota(jnp.int32, (E, BT), 0)
    win = jnp.min(jnp.where(soft == ms, ii, E), axis=0, keepdims=True)
    hard_ref[...] = jnp.where(ii == win, 1.0, 0.0).astype(jnp.float32)


def kernel(x_z, W1, b1, W2, b2, W3, b3, temperature):
    x2d = x_z.reshape(T * E, D)
    b1r = b1.reshape(1, H)
    b2r = b2.reshape(1, H)
    scal = jnp.stack([b3[0], 1.0 / temperature]).reshape(1, 2).astype(jnp.float32)

    R = BT * E
    soft, hard = pl.pallas_call(
        _gate_kernel,
        grid=(T // BT,),
        in_specs=[
            pl.BlockSpec((R, D), lambda i: (i, 0)),
            pl.BlockSpec((D, H), lambda i: (0, 0)),
            pl.BlockSpec((1, H), lambda i: (0, 0)),
            pl.BlockSpec((H, H), lambda i: (0, 0)),
            pl.BlockSpec((1, H), lambda i: (0, 0)),
            pl.BlockSpec((H, 1), lambda i: (0, 0)),
            pl.BlockSpec((1, 2), lambda i: (0, 0)),
        ],
        out_specs=[
            pl.BlockSpec((E, BT), lambda i: (0, i)),
            pl.BlockSpec((E, BT), lambda i: (0, i)),
        ],
        out_shape=[
            jax.ShapeDtypeStruct((E, T), jnp.float32),
            jax.ShapeDtypeStruct((E, T), jnp.float32),
        ],
    )(x2d, W1, b1r, W2, b2r, W3, scal)
    return soft.T[..., None], hard.T[..., None]
